# baseline (speedup 1.0000x reference)
import functools

import jax
import jax.numpy as jnp
from jax import lax
from jax.experimental import pallas as pl
from jax.experimental.pallas import tpu as pltpu
from jax.experimental.pallas import tpu_sc as plsc

_EPS = 1e-12
_ROWS = 68229
_NW = 32
_CHUNK = 2136
_ROWS_PAD = _NW * _CHUNK
_LAST_BASE = (_NW - 1) * _CHUNK
_LAST_CHUNK = _ROWS - _LAST_BASE
_M = _ROWS_PAD * 8 // 128
_BLK = 712
_TC_GRID = _M // _BLK


def _sc_compact(x_hbm, t_hbm, xs_hbm, ts_hbm, xv, tv, xl, tl):
    cid = lax.axis_index("c")
    sid = lax.axis_index("s")
    wid = sid * 2 + cid
    base = wid * _CHUNK

    @pl.when(wid < _NW - 1)
    def _full():
        pltpu.sync_copy(x_hbm.at[pl.ds(base, _CHUNK), pl.ds(0, 8)], xv)
        pltpu.sync_copy(t_hbm.at[pl.ds(base, _CHUNK), pl.ds(0, 8)], tv)
        pltpu.sync_copy(xv, xs_hbm.at[pl.ds(base, _CHUNK), :])
        pltpu.sync_copy(tv, ts_hbm.at[pl.ds(base, _CHUNK), :])

    @pl.when(wid == _NW - 1)
    def _tail():
        pltpu.sync_copy(
            x_hbm.at[pl.ds(_LAST_BASE, _LAST_CHUNK), pl.ds(0, 8)], xl)
        pltpu.sync_copy(
            t_hbm.at[pl.ds(_LAST_BASE, _LAST_CHUNK), pl.ds(0, 8)], tl)
        pltpu.sync_copy(xl, xs_hbm.at[pl.ds(_LAST_BASE, _LAST_CHUNK), :])
        pltpu.sync_copy(tl, ts_hbm.at[pl.ds(_LAST_BASE, _LAST_CHUNK), :])


def _tc_loss(xs_ref, ts_ref, out_ref, acc_ref):
    i = pl.program_id(0)

    @pl.when(i == 0)
    def _init():
        acc_ref[0] = 0.0
        acc_ref[1] = 0.0

    xb = xs_ref[...]
    tb = ts_ref[...]

    lane = lax.broadcasted_iota(jnp.int32, (_BLK, 128), 1)
    cls = lane % 8
    vrow = lax.broadcasted_iota(jnp.int32, (_BLK, 128), 0) + i * _BLK
    grow = vrow * 16 + lane // 8
    valid = grow < _ROWS

    obj01 = jnp.where((tb > 0.0) & (cls == 4) & valid, 1.0, 0.0)
    li = lax.broadcasted_iota(jnp.int32, (128, 128), 0)
    lj = lax.broadcasted_iota(jnp.int32, (128, 128), 1)
    bmat = jnp.where((li // 8 == lj // 8) & (li % 8 == 4), 1.0, 0.0)
    objb = jnp.dot(obj01, bmat, preferred_element_type=jnp.float32)

    p = jnp.clip(xb, _EPS, 1.0 - _EPS)
    elem = -(tb * jnp.log(p) + (1.0 - tb) * jnp.log(1.0 - p))
    take = (objb > 0.0) & (cls < 4)
    acc_ref[0] += jnp.sum(jnp.where(take, elem, 0.0))
    acc_ref[1] += jnp.sum(obj01)

    @pl.when(i == _TC_GRID - 1)
    def _fin():
        denom = jnp.maximum(acc_ref[1] * 2.0, 1.0)
        out_ref[...] = jnp.full((1, 1), acc_ref[0] / denom, jnp.float32)


def _compact_sc(xf, tf):
    mesh = plsc.VectorSubcoreMesh(core_axis_name="c", subcore_axis_name="s")
    f = pl.kernel(
        _sc_compact,
        mesh=mesh,
        compiler_params=pltpu.CompilerParams(use_tc_tiling_on_sc=False),
        out_type=[
            jax.ShapeDtypeStruct((_ROWS_PAD, 8), jnp.float32),
            jax.ShapeDtypeStruct((_ROWS_PAD, 8), jnp.float32),
        ],
        scratch_types=[
            pltpu.VMEM((_CHUNK, 8), jnp.float32),
            pltpu.VMEM((_CHUNK, 8), jnp.float32),
            pltpu.VMEM((_LAST_CHUNK, 8), jnp.float32),
            pltpu.VMEM((_LAST_CHUNK, 8), jnp.float32),
        ],
    )
    return f(xf, tf)


def kernel(x, target):
    b, n, c = x.shape
    xf = x.reshape(b * n, c)
    tf = target.reshape(b * n, c)

    xs, ts = _compact_sc(xf, tf)
    xsv = xs.reshape(_M, 128)
    tsv = ts.reshape(_M, 128)

    spec = pl.BlockSpec((_BLK, 128), lambda i: (i, 0))
    out = pl.pallas_call(
        _tc_loss,
        grid=(_TC_GRID,),
        in_specs=[spec, spec],
        out_specs=pl.BlockSpec((1, 1), lambda i: (0, 0)),
        out_shape=jax.ShapeDtypeStruct((1, 1), jnp.float32),
        scratch_shapes=[pltpu.SMEM((2,), jnp.float32)],
    )(xsv, tsv)
    return out[0, 0]
